# Initial kernel scaffold; baseline (speedup 1.0000x reference)
import jax
import jax.numpy as jnp
from jax import lax
from jax.experimental import pallas as pl
from jax.experimental.pallas import tpu as pltpu
from jax.experimental.pallas import tpu_sc as plsc

N = 10000
E = 320000
D_IN = 128
D_OUT = 64
G = 64

NC = 2
NS = 16
NW = NC * NS
EPW = E // NW
CHUNK = 80
NCHUNK = EPW // CHUNK
NPAD = 10240
RPT = NPAD // NS
BLK = 1000
NBLK = N // BLK

_MESH = plsc.VectorSubcoreMesh(core_axis_name="c", subcore_axis_name="s")


def _deg_body(dst_hbm, ew_hbm, deg_out, deg_sh, dst_v, ew_v, zb):
    c = lax.axis_index("c")
    s = lax.axis_index("s")
    wid = c * NS + s

    z = jnp.zeros((16,), jnp.float32)

    @pl.loop(0, RPT // 16)
    def _(i):
        zb[pl.ds(i * 16, 16)] = z

    pltpu.sync_copy(zb, deg_sh.at[pl.ds(s * RPT, RPT)])
    plsc.subcore_barrier()

    pltpu.sync_copy(dst_hbm.at[wid], dst_v)
    pltpu.sync_copy(ew_hbm.at[wid], ew_v)

    @pl.loop(0, NCHUNK)
    def _(j):
        pltpu.sync_copy(ew_v.at[j], deg_sh.at[dst_v.at[j]], add=True)

    plsc.subcore_barrier()
    pltpu.sync_copy(deg_sh.at[pl.ds(s * RPT, RPT)], deg_out.at[c, pl.ds(s * RPT, RPT)])


_deg_kernel = pl.kernel(
    _deg_body,
    out_type=jax.ShapeDtypeStruct((NC, NPAD), jnp.float32),
    mesh=_MESH,
    scratch_types=[
        pltpu.VMEM_SHARED((NPAD,), jnp.float32),
        pltpu.VMEM((NCHUNK, CHUNK), jnp.int32),
        pltpu.VMEM((NCHUNK, CHUNK), jnp.float32),
        pltpu.VMEM((RPT,), jnp.float32),
    ],
)


def _msg_body(xw_hbm, deg_hbm, src_hbm, dst_hbm, ew_hbm, eh_out, dinv_out,
              h_sh, dinv_v, dga, src_v, dst_v, ew_v, scal, rows, sem):
    c = lax.axis_index("c")
    s = lax.axis_index("s")
    wid = c * NS + s

    pltpu.sync_copy(deg_hbm.at[0], dinv_v)
    pltpu.sync_copy(deg_hbm.at[1], dga)

    @pl.loop(0, NPAD // 16)
    def _(i):
        sl = pl.ds(i * 16, 16)
        d = dinv_v[sl] + dga[sl] + 1.0
        di = lax.bitcast_convert_type(d, jnp.int32)
        yi = jnp.int32(0x5F3759DF) - lax.shift_right_logical(di, 1)
        y = lax.bitcast_convert_type(yi, jnp.float32)
        y = y * (1.5 - 0.5 * d * y * y)
        y = y * (1.5 - 0.5 * d * y * y)
        y = y * (1.5 - 0.5 * d * y * y)
        dinv_v[sl] = y

    @pl.when(c == 0)
    def _():
        pltpu.sync_copy(dinv_v.at[pl.ds(s * RPT, RPT)],
                        dinv_out.at[pl.ds(s * RPT, RPT)])

    z = jnp.zeros((16,), jnp.float32)

    @pl.loop(0, CHUNK)
    def _(r):
        for dd in range(4):
            rows[r, pl.ds(dd * 16, 16)] = z

    @pl.loop(0, RPT // CHUNK)
    def _(k):
        pltpu.sync_copy(rows, h_sh.at[pl.ds(s * RPT + k * CHUNK, CHUNK)])

    plsc.subcore_barrier()

    pltpu.sync_copy(src_hbm.at[wid], src_v)
    pltpu.sync_copy(dst_hbm.at[wid], dst_v)
    pltpu.sync_copy(ew_hbm.at[wid], ew_v)

    @pl.loop(0, NCHUNK)
    def _(j):
        pltpu.async_copy(xw_hbm.at[src_v.at[j]], rows, sem).wait()
        for g in range(CHUNK // 16):
            sl = pl.ds(g * 16, 16)
            iv = src_v[j, sl]
            wv = ew_v[j, sl]
            dv = plsc.load_gather(dinv_v, [iv])
            scal[sl] = wv * dv

        @pl.loop(0, CHUNK)
        def _(e):
            sc_ = scal[e]
            for dd in range(4):
                sl2 = pl.ds(dd * 16, 16)
                rows[e, sl2] = rows[e, sl2] * sc_

        pltpu.sync_copy(rows, h_sh.at[dst_v.at[j]], add=True)

    plsc.subcore_barrier()
    pltpu.sync_copy(h_sh.at[pl.ds(s * RPT, RPT)], eh_out.at[c, pl.ds(s * RPT, RPT)])


_msg_kernel = pl.kernel(
    _msg_body,
    out_type=(
        jax.ShapeDtypeStruct((NC, NPAD, D_OUT), jnp.float32),
        jax.ShapeDtypeStruct((NPAD,), jnp.float32),
    ),
    mesh=_MESH,
    scratch_types=[
        pltpu.VMEM_SHARED((NPAD, D_OUT), jnp.float32),
        pltpu.VMEM((NPAD,), jnp.float32),
        pltpu.VMEM((NPAD,), jnp.float32),
        pltpu.VMEM((NCHUNK, CHUNK), jnp.int32),
        pltpu.VMEM((NCHUNK, CHUNK), jnp.int32),
        pltpu.VMEM((NCHUNK, CHUNK), jnp.float32),
        pltpu.VMEM((CHUNK,), jnp.float32),
        pltpu.VMEM((CHUNK, D_OUT), jnp.float32),
        pltpu.SemaphoreType.DMA,
    ],
)


def _mm_body(x_ref, w_ref, o_ref):
    o_ref[...] = jnp.dot(x_ref[...], w_ref[...], preferred_element_type=jnp.float32)


_mm_kernel = pl.pallas_call(
    _mm_body,
    out_shape=jax.ShapeDtypeStruct((N, D_OUT), jnp.float32),
)


def _fin_body(eh_ref, xw_ref, dinv_ref, bat_ref, bc_ref, wl_ref, bl_ref,
              o_ref, sums, counts):
    i = pl.program_id(0)

    @pl.when(i == 0)
    def _():
        sums[...] = jnp.zeros_like(sums)
        counts[...] = jnp.zeros_like(counts)

    dv = dinv_ref[...]
    t = dv * (eh_ref[0] + eh_ref[1] + dv * xw_ref[...]) + bc_ref[...]
    t = jnp.maximum(t, 0.0)
    oh = (bat_ref[...] == lax.broadcasted_iota(jnp.int32, (BLK, G), 1))
    oh = oh.astype(jnp.float32)
    sums[...] += lax.dot_general(oh, t, (((0,), (0,)), ((), ())),
                                 preferred_element_type=jnp.float32)
    counts[0, :] += jnp.sum(oh, axis=0)

    @pl.when(i == NBLK - 1)
    def _():
        pooled = sums[...] / jnp.maximum(counts[0, :], 1.0)[:, None]
        logits = jnp.dot(pooled, wl_ref[...], preferred_element_type=jnp.float32)
        o_ref[...] = jax.nn.sigmoid(logits + bl_ref[...])


_fin_kernel = pl.pallas_call(
    _fin_body,
    grid=(NBLK,),
    in_specs=[
        pl.BlockSpec((NC, BLK, D_OUT), lambda i: (0, i, 0)),
        pl.BlockSpec((BLK, D_OUT), lambda i: (i, 0)),
        pl.BlockSpec((BLK, 1), lambda i: (i, 0)),
        pl.BlockSpec((BLK, 1), lambda i: (i, 0)),
        pl.BlockSpec((1, D_OUT), lambda i: (0, 0)),
        pl.BlockSpec((D_OUT, 1), lambda i: (0, 0)),
        pl.BlockSpec((1, 1), lambda i: (0, 0)),
    ],
    out_specs=pl.BlockSpec((G, 1), lambda i: (0, 0)),
    out_shape=jax.ShapeDtypeStruct((G, 1), jnp.float32),
    scratch_shapes=[
        pltpu.VMEM((G, D_OUT), jnp.float32),
        pltpu.VMEM((8, D_OUT), jnp.float32),
    ],
)


def kernel(x, edge_index, edge_attr, batch, W_conv, b_conv, W_lin, b_lin):
    src = edge_index[0].reshape(NW, NCHUNK, CHUNK)
    dst = edge_index[1].reshape(NW, NCHUNK, CHUNK)
    ew = edge_attr.reshape(NW, NCHUNK, CHUNK)

    xw = _mm_kernel(x, W_conv)
    deg_parts = _deg_kernel(dst, ew)
    eh, dinv = _msg_kernel(xw, deg_parts, src, dst, ew)

    out = _fin_kernel(
        eh, xw, dinv.reshape(NPAD, 1)[:N], batch.reshape(N, 1),
        b_conv.reshape(1, D_OUT), W_lin, b_lin.reshape(1, 1),
    )
    return out


# speedup vs baseline: 27.3760x; 27.3760x over previous
import jax
import jax.numpy as jnp
from jax import lax
from jax.experimental import pallas as pl
from jax.experimental.pallas import tpu as pltpu
from jax.experimental.pallas import tpu_sc as plsc

N = 10000
E = 320000
D_IN = 128
D_OUT = 64
G = 64

NC = 2
NS = 16
NW = NC * NS
EPW = E // NW
CHUNK = 80
NCHUNK = EPW // CHUNK
NPAD = 10240
RPT = NPAD // NS
BLK = 1000
NBLK = N // BLK

_MESH = plsc.VectorSubcoreMesh(core_axis_name="c", subcore_axis_name="s")


def _deg_body(dst_hbm, ew_hbm, deg_out, deg_sh, dst_v, ew_v, zb):
    c = lax.axis_index("c")
    s = lax.axis_index("s")
    wid = c * NS + s

    z = jnp.zeros((16,), jnp.float32)

    @pl.loop(0, RPT // 16)
    def _(i):
        zb[pl.ds(i * 16, 16)] = z

    pltpu.sync_copy(zb, deg_sh.at[pl.ds(s * RPT, RPT)])
    plsc.subcore_barrier()

    pltpu.sync_copy(dst_hbm.at[wid], dst_v)
    pltpu.sync_copy(ew_hbm.at[wid], ew_v)

    @pl.loop(0, NCHUNK)
    def _(j):
        pltpu.sync_copy(ew_v.at[j], deg_sh.at[dst_v.at[j]], add=True)

    plsc.subcore_barrier()
    pltpu.sync_copy(deg_sh.at[pl.ds(s * RPT, RPT)], deg_out.at[c, pl.ds(s * RPT, RPT)])


_deg_kernel = pl.kernel(
    _deg_body,
    out_type=jax.ShapeDtypeStruct((NC, NPAD), jnp.float32),
    mesh=_MESH,
    scratch_types=[
        pltpu.VMEM_SHARED((NPAD,), jnp.float32),
        pltpu.VMEM((NCHUNK, CHUNK), jnp.int32),
        pltpu.VMEM((NCHUNK, CHUNK), jnp.float32),
        pltpu.VMEM((RPT,), jnp.float32),
    ],
)


def _msg_body(xw_hbm, deg_hbm, src_hbm, dst_hbm, ew_hbm, eh_out, dinv_out,
              h_sh, dinv_v, dga, src_v, dst_v, ew_v, rows, sem):
    c = lax.axis_index("c")
    s = lax.axis_index("s")
    wid = c * NS + s

    pltpu.sync_copy(deg_hbm.at[0], dinv_v)
    pltpu.sync_copy(deg_hbm.at[1], dga)

    @pl.loop(0, NPAD // 16)
    def _(i):
        sl = pl.ds(i * 16, 16)
        d = dinv_v[sl] + dga[sl] + 1.0
        di = lax.bitcast_convert_type(d, jnp.int32)
        yi = jnp.int32(0x5F3759DF) - lax.shift_right_logical(di, 1)
        y = lax.bitcast_convert_type(yi, jnp.float32)
        y = y * (1.5 - 0.5 * d * y * y)
        y = y * (1.5 - 0.5 * d * y * y)
        y = y * (1.5 - 0.5 * d * y * y)
        dinv_v[sl] = y

    @pl.when(c == 0)
    def _():
        pltpu.sync_copy(dinv_v.at[pl.ds(s * RPT, RPT)],
                        dinv_out.at[pl.ds(s * RPT, RPT)])

    z = jnp.zeros((16,), jnp.float32)

    @pl.loop(0, CHUNK)
    def _(r):
        for dd in range(4):
            rows[r, pl.ds(dd * 16, 16)] = z

    @pl.loop(0, RPT // CHUNK)
    def _(k):
        pltpu.sync_copy(rows, h_sh.at[pl.ds(s * RPT + k * CHUNK, CHUNK)])

    plsc.subcore_barrier()

    pltpu.sync_copy(src_hbm.at[wid], src_v)
    pltpu.sync_copy(dst_hbm.at[wid], dst_v)
    pltpu.sync_copy(ew_hbm.at[wid], ew_v)

    @pl.loop(0, NCHUNK)
    def _(j):
        pltpu.async_copy(xw_hbm.at[src_v.at[j]], rows, sem).wait()
        for g in range(CHUNK // 16):
            sl = pl.ds(g * 16, 16)
            iv = src_v[j, sl]
            wv = ew_v[j, sl]
            dv = plsc.load_gather(dinv_v, [iv])
            sv = wv * dv
            for l in range(16):
                e = g * 16 + l
                s_ = sv[l]
                for dd in range(4):
                    sl2 = pl.ds(dd * 16, 16)
                    rows[e, sl2] = rows[e, sl2] * s_

        pltpu.sync_copy(rows, h_sh.at[dst_v.at[j]], add=True)

    plsc.subcore_barrier()
    pltpu.sync_copy(h_sh.at[pl.ds(s * RPT, RPT)], eh_out.at[c, pl.ds(s * RPT, RPT)])


_msg_kernel = pl.kernel(
    _msg_body,
    out_type=(
        jax.ShapeDtypeStruct((NC, NPAD, D_OUT), jnp.float32),
        jax.ShapeDtypeStruct((NPAD,), jnp.float32),
    ),
    mesh=_MESH,
    compiler_params=pltpu.CompilerParams(needs_layout_passes=False,
                                         use_tc_tiling_on_sc=False),
    scratch_types=[
        pltpu.VMEM_SHARED((NPAD, D_OUT), jnp.float32),
        pltpu.VMEM((NPAD,), jnp.float32),
        pltpu.VMEM((NPAD,), jnp.float32),
        pltpu.VMEM((NCHUNK, CHUNK), jnp.int32),
        pltpu.VMEM((NCHUNK, CHUNK), jnp.int32),
        pltpu.VMEM((NCHUNK, CHUNK), jnp.float32),
        pltpu.VMEM((CHUNK, D_OUT), jnp.float32),
        pltpu.SemaphoreType.DMA,
    ],
)


def _mm_body(x_ref, w_ref, o_ref):
    o_ref[...] = jnp.dot(x_ref[...], w_ref[...], preferred_element_type=jnp.float32)


_mm_kernel = pl.pallas_call(
    _mm_body,
    out_shape=jax.ShapeDtypeStruct((N, D_OUT), jnp.float32),
)


def _fin_body(eh_ref, xw_ref, dinv_ref, bat_ref, bc_ref, wl_ref, bl_ref,
              o_ref, sums, counts):
    i = pl.program_id(0)

    @pl.when(i == 0)
    def _():
        sums[...] = jnp.zeros_like(sums)
        counts[...] = jnp.zeros_like(counts)

    dv = dinv_ref[...]
    t = dv * (eh_ref[0] + eh_ref[1] + dv * xw_ref[...]) + bc_ref[...]
    t = jnp.maximum(t, 0.0)
    oh = (bat_ref[...] == lax.broadcasted_iota(jnp.int32, (BLK, G), 1))
    oh = oh.astype(jnp.float32)
    sums[...] += lax.dot_general(oh, t, (((0,), (0,)), ((), ())),
                                 preferred_element_type=jnp.float32)
    counts[0, :] += jnp.sum(oh, axis=0)

    @pl.when(i == NBLK - 1)
    def _():
        pooled = sums[...] / jnp.maximum(counts[0, :], 1.0)[:, None]
        logits = jnp.dot(pooled, wl_ref[...], preferred_element_type=jnp.float32)
        o_ref[...] = jax.nn.sigmoid(logits + bl_ref[...])


_fin_kernel = pl.pallas_call(
    _fin_body,
    grid=(NBLK,),
    in_specs=[
        pl.BlockSpec((NC, BLK, D_OUT), lambda i: (0, i, 0)),
        pl.BlockSpec((BLK, D_OUT), lambda i: (i, 0)),
        pl.BlockSpec((BLK, 1), lambda i: (i, 0)),
        pl.BlockSpec((BLK, 1), lambda i: (i, 0)),
        pl.BlockSpec((1, D_OUT), lambda i: (0, 0)),
        pl.BlockSpec((D_OUT, 1), lambda i: (0, 0)),
        pl.BlockSpec((1, 1), lambda i: (0, 0)),
    ],
    out_specs=pl.BlockSpec((G, 1), lambda i: (0, 0)),
    out_shape=jax.ShapeDtypeStruct((G, 1), jnp.float32),
    scratch_shapes=[
        pltpu.VMEM((G, D_OUT), jnp.float32),
        pltpu.VMEM((8, D_OUT), jnp.float32),
    ],
)


def kernel(x, edge_index, edge_attr, batch, W_conv, b_conv, W_lin, b_lin):
    src = edge_index[0].reshape(NW, NCHUNK, CHUNK)
    dst = edge_index[1].reshape(NW, NCHUNK, CHUNK)
    ew = edge_attr.reshape(NW, NCHUNK, CHUNK)

    xw = _mm_kernel(x, W_conv)
    deg_parts = _deg_kernel(dst, ew)
    eh, dinv = _msg_kernel(xw, deg_parts, src, dst, ew)

    out = _fin_kernel(
        eh, xw, dinv.reshape(NPAD, 1)[:N], batch.reshape(N, 1),
        b_conv.reshape(1, D_OUT), W_lin, b_lin.reshape(1, 1),
    )
    return out


# speedup vs baseline: 40.1496x; 1.4666x over previous
import jax
import jax.numpy as jnp
from jax import lax
from jax.experimental import pallas as pl
from jax.experimental.pallas import tpu as pltpu
from jax.experimental.pallas import tpu_sc as plsc

N = 10000
E = 320000
D_IN = 128
D_OUT = 64
G = 64

NC = 2
NS = 16
NW = NC * NS
EPW = E // NW
CHUNK = 80
NCHUNK = EPW // CHUNK
NPAD = 10240
RPT = NPAD // NS
BLK = 1000
NBLK = N // BLK

_MESH = plsc.VectorSubcoreMesh(core_axis_name="c", subcore_axis_name="s")


def _deg_body(dst_hbm, ew_hbm, deg_out, deg_sh, dst_v, ew_v, zb):
    c = lax.axis_index("c")
    s = lax.axis_index("s")
    wid = c * NS + s

    z = jnp.zeros((16,), jnp.float32)

    @pl.loop(0, RPT // 16)
    def _(i):
        zb[pl.ds(i * 16, 16)] = z

    pltpu.sync_copy(zb, deg_sh.at[pl.ds(s * RPT, RPT)])
    plsc.subcore_barrier()

    pltpu.sync_copy(dst_hbm.at[wid], dst_v)
    pltpu.sync_copy(ew_hbm.at[wid], ew_v)

    @pl.loop(0, NCHUNK)
    def _(j):
        pltpu.sync_copy(ew_v.at[j], deg_sh.at[dst_v.at[j]], add=True)

    plsc.subcore_barrier()
    pltpu.sync_copy(deg_sh.at[pl.ds(s * RPT, RPT)], deg_out.at[c, pl.ds(s * RPT, RPT)])


_deg_kernel = pl.kernel(
    _deg_body,
    out_type=jax.ShapeDtypeStruct((NC, NPAD), jnp.float32),
    mesh=_MESH,
    scratch_types=[
        pltpu.VMEM_SHARED((NPAD,), jnp.float32),
        pltpu.VMEM((NCHUNK, CHUNK), jnp.int32),
        pltpu.VMEM((NCHUNK, CHUNK), jnp.float32),
        pltpu.VMEM((RPT,), jnp.float32),
    ],
)


def _msg_body(xw_hbm, deg_hbm, src_hbm, dst_hbm, ew_hbm, eh_out, dinv_out,
              h_sh, dinv_v, dga, src_v, dst_v, ew_v,
              ab0, ab1, sb0, sb1, gs0, gs1, ss0, ss1):
    abufs = (ab0, ab1)
    sbufs = (sb0, sb1)
    gsems = (gs0, gs1)
    ssems = (ss0, ss1)
    c = lax.axis_index("c")
    s = lax.axis_index("s")
    wid = c * NS + s

    pltpu.sync_copy(deg_hbm.at[0], dinv_v)
    pltpu.sync_copy(deg_hbm.at[1], dga)

    @pl.loop(0, NPAD // 16)
    def _(i):
        sl = pl.ds(i * 16, 16)
        d = dinv_v[sl] + dga[sl] + 1.0
        di = lax.bitcast_convert_type(d, jnp.int32)
        yi = jnp.int32(0x5F3759DF) - lax.shift_right_logical(di, 1)
        y = lax.bitcast_convert_type(yi, jnp.float32)
        y = y * (1.5 - 0.5 * d * y * y)
        y = y * (1.5 - 0.5 * d * y * y)
        y = y * (1.5 - 0.5 * d * y * y)
        dinv_v[sl] = y

    @pl.when(c == 0)
    def _():
        pltpu.sync_copy(dinv_v.at[pl.ds(s * RPT, RPT)],
                        dinv_out.at[pl.ds(s * RPT, RPT)])

    z = jnp.zeros((16,), jnp.float32)

    @pl.loop(0, CHUNK)
    def _(r):
        for dd in range(4):
            sb0[r, pl.ds(dd * 16, 16)] = z

    @pl.loop(0, RPT // CHUNK)
    def _(k):
        pltpu.sync_copy(sb0, h_sh.at[pl.ds(s * RPT + k * CHUNK, CHUNK)])

    plsc.subcore_barrier()

    pltpu.sync_copy(src_hbm.at[wid], src_v)
    pltpu.sync_copy(dst_hbm.at[wid], dst_v)
    pltpu.sync_copy(ew_hbm.at[wid], ew_v)

    def fire_gather(j, b):
        pltpu.async_copy(xw_hbm.at[src_v.at[j]], abufs[b], gsems[b])

    def wait_gather(j, b):
        pltpu.make_async_copy(xw_hbm.at[src_v.at[j]], abufs[b], gsems[b]).wait()

    def fire_scatter(j, b):
        pltpu.async_copy(sbufs[b], h_sh.at[dst_v.at[j]], ssems[b], add=True)

    def wait_scatter(j, b):
        pltpu.make_async_copy(sbufs[b], h_sh.at[dst_v.at[j]], ssems[b]).wait()

    def scale(j, b):
        ab = abufs[b]
        sb = sbufs[b]
        for g in range(CHUNK // 16):
            sl = pl.ds(g * 16, 16)
            iv = src_v[j, sl]
            wv = ew_v[j, sl]
            dv = plsc.load_gather(dinv_v, [iv])
            sv = wv * dv
            for l in range(16):
                e = g * 16 + l
                s_ = sv[l]
                for dd in range(4):
                    sl2 = pl.ds(dd * 16, 16)
                    sb[e, sl2] = ab[e, sl2] * s_

    fire_gather(0, 0)
    fire_gather(1, 1)

    @pl.loop(0, NCHUNK // 2)
    def _(jj):
        for b in range(2):
            j = 2 * jj + b
            wait_gather(j, b)

            @pl.when(jj > 0)
            def _():
                wait_scatter(j - 2, b)

            scale(j, b)
            fire_scatter(j, b)

            @pl.when(j + 2 < NCHUNK)
            def _():
                fire_gather(j + 2, b)

    jl = NCHUNK - 1
    wait_gather(jl, 0)
    wait_scatter(jl - 2, 0)
    scale(jl, 0)
    fire_scatter(jl, 0)
    wait_scatter(jl - 1, 1)
    wait_scatter(jl, 0)
    plsc.subcore_barrier()
    pltpu.sync_copy(h_sh.at[pl.ds(s * RPT, RPT)], eh_out.at[c, pl.ds(s * RPT, RPT)])


_msg_kernel = pl.kernel(
    _msg_body,
    out_type=(
        jax.ShapeDtypeStruct((NC, NPAD, D_OUT), jnp.float32),
        jax.ShapeDtypeStruct((NPAD,), jnp.float32),
    ),
    mesh=_MESH,
    compiler_params=pltpu.CompilerParams(needs_layout_passes=False,
                                         use_tc_tiling_on_sc=False),
    scratch_types=[
        pltpu.VMEM_SHARED((NPAD, D_OUT), jnp.float32),
        pltpu.VMEM((NPAD,), jnp.float32),
        pltpu.VMEM((NPAD,), jnp.float32),
        pltpu.VMEM((NCHUNK, CHUNK), jnp.int32),
        pltpu.VMEM((NCHUNK, CHUNK), jnp.int32),
        pltpu.VMEM((NCHUNK, CHUNK), jnp.float32),
        pltpu.VMEM((CHUNK, D_OUT), jnp.float32),
        pltpu.VMEM((CHUNK, D_OUT), jnp.float32),
        pltpu.VMEM((CHUNK, D_OUT), jnp.float32),
        pltpu.VMEM((CHUNK, D_OUT), jnp.float32),
        pltpu.SemaphoreType.DMA,
        pltpu.SemaphoreType.DMA,
        pltpu.SemaphoreType.DMA,
        pltpu.SemaphoreType.DMA,
    ],
)


def _mm_body(x_ref, w_ref, o_ref):
    o_ref[...] = jnp.dot(x_ref[...], w_ref[...], preferred_element_type=jnp.float32)


_mm_kernel = pl.pallas_call(
    _mm_body,
    out_shape=jax.ShapeDtypeStruct((N, D_OUT), jnp.float32),
)


def _fin_body(eh_ref, xw_ref, dinv_ref, bat_ref, bc_ref, wl_ref, bl_ref,
              o_ref, sums, counts):
    i = pl.program_id(0)

    @pl.when(i == 0)
    def _():
        sums[...] = jnp.zeros_like(sums)
        counts[...] = jnp.zeros_like(counts)

    dv = dinv_ref[...]
    t = dv * (eh_ref[0] + eh_ref[1] + dv * xw_ref[...]) + bc_ref[...]
    t = jnp.maximum(t, 0.0)
    oh = (bat_ref[...] == lax.broadcasted_iota(jnp.int32, (BLK, G), 1))
    oh = oh.astype(jnp.float32)
    sums[...] += lax.dot_general(oh, t, (((0,), (0,)), ((), ())),
                                 preferred_element_type=jnp.float32)
    counts[0, :] += jnp.sum(oh, axis=0)

    @pl.when(i == NBLK - 1)
    def _():
        pooled = sums[...] / jnp.maximum(counts[0, :], 1.0)[:, None]
        logits = jnp.dot(pooled, wl_ref[...], preferred_element_type=jnp.float32)
        o_ref[...] = jax.nn.sigmoid(logits + bl_ref[...])


_fin_kernel = pl.pallas_call(
    _fin_body,
    grid=(NBLK,),
    in_specs=[
        pl.BlockSpec((NC, BLK, D_OUT), lambda i: (0, i, 0)),
        pl.BlockSpec((BLK, D_OUT), lambda i: (i, 0)),
        pl.BlockSpec((BLK, 1), lambda i: (i, 0)),
        pl.BlockSpec((BLK, 1), lambda i: (i, 0)),
        pl.BlockSpec((1, D_OUT), lambda i: (0, 0)),
        pl.BlockSpec((D_OUT, 1), lambda i: (0, 0)),
        pl.BlockSpec((1, 1), lambda i: (0, 0)),
    ],
    out_specs=pl.BlockSpec((G, 1), lambda i: (0, 0)),
    out_shape=jax.ShapeDtypeStruct((G, 1), jnp.float32),
    scratch_shapes=[
        pltpu.VMEM((G, D_OUT), jnp.float32),
        pltpu.VMEM((8, D_OUT), jnp.float32),
    ],
)


def kernel(x, edge_index, edge_attr, batch, W_conv, b_conv, W_lin, b_lin):
    src = edge_index[0].reshape(NW, NCHUNK, CHUNK)
    dst = edge_index[1].reshape(NW, NCHUNK, CHUNK)
    ew = edge_attr.reshape(NW, NCHUNK, CHUNK)

    xw = _mm_kernel(x, W_conv)
    deg_parts = _deg_kernel(dst, ew)
    eh, dinv = _msg_kernel(xw, deg_parts, src, dst, ew)

    out = _fin_kernel(
        eh, xw, dinv.reshape(NPAD, 1)[:N], batch.reshape(N, 1),
        b_conv.reshape(1, D_OUT), W_lin, b_lin.reshape(1, 1),
    )
    return out


# speedup vs baseline: 44.1434x; 1.0995x over previous
import jax
import jax.numpy as jnp
from jax import lax
from jax.experimental import pallas as pl
from jax.experimental.pallas import tpu as pltpu
from jax.experimental.pallas import tpu_sc as plsc

N = 10000
E = 320000
D_IN = 128
D_OUT = 64
G = 64

NC = 2
NS = 16
NW = NC * NS
EPW = E // NW
CHUNK = 80
NCHUNK = EPW // CHUNK
NPAD = 10240
RPT = NPAD // NS
BLK = 1000
NBLK = N // BLK

_MESH = plsc.VectorSubcoreMesh(core_axis_name="c", subcore_axis_name="s")


def _msg_body(xw_hbm, src_hbm, dst_hbm, ew_hbm, eh_out, dinv_out,
              h_sh, deg_sh, dinv_v, dga, src_v, dst_v, ew_v, zb,
              ab0, ab1, sb0, sb1, gs0, gs1, ss0, ss1):
    abufs = (ab0, ab1)
    sbufs = (sb0, sb1)
    gsems = (gs0, gs1)
    ssems = (ss0, ss1)
    c = lax.axis_index("c")
    s = lax.axis_index("s")
    wid = c * NS + s

    z = jnp.zeros((16,), jnp.float32)

    @pl.loop(0, RPT // 16)
    def _(i):
        zb[pl.ds(i * 16, 16)] = z

    pltpu.sync_copy(zb, deg_sh.at[pl.ds(s * RPT, RPT)])

    @pl.loop(0, CHUNK)
    def _(r):
        for dd in range(4):
            sb0[r, pl.ds(dd * 16, 16)] = z

    @pl.loop(0, RPT // CHUNK)
    def _(k):
        pltpu.sync_copy(sb0, h_sh.at[pl.ds(s * RPT + k * CHUNK, CHUNK)])

    plsc.subcore_barrier()

    for q in range(2):
        w2 = 2 * s + q
        pltpu.sync_copy(dst_hbm.at[w2], dst_v)
        pltpu.sync_copy(ew_hbm.at[w2], ew_v)

        @pl.loop(0, NCHUNK // 25)
        def _(blk):
            @pl.loop(0, 25)
            def _(i):
                j = blk * 25 + i
                pltpu.async_copy(ew_v.at[j], deg_sh.at[dst_v.at[j]], ss0,
                                 add=True)

            @pl.loop(0, 25)
            def _(i):
                j = blk * 25 + i
                pltpu.make_async_copy(ew_v.at[j], deg_sh.at[dst_v.at[j]],
                                      ss0).wait()

    plsc.subcore_barrier()

    pltpu.sync_copy(deg_sh, dga)

    @pl.loop(0, NPAD // 16)
    def _(i):
        sl = pl.ds(i * 16, 16)
        d = dga[sl] + 1.0
        di = lax.bitcast_convert_type(d, jnp.int32)
        yi = jnp.int32(0x5F3759DF) - lax.shift_right_logical(di, 1)
        y = lax.bitcast_convert_type(yi, jnp.float32)
        y = y * (1.5 - 0.5 * d * y * y)
        y = y * (1.5 - 0.5 * d * y * y)
        y = y * (1.5 - 0.5 * d * y * y)
        dinv_v[sl] = y

    @pl.when(c == 0)
    def _():
        pltpu.sync_copy(dinv_v.at[pl.ds(s * RPT, RPT)],
                        dinv_out.at[pl.ds(s * RPT, RPT)])

    pltpu.sync_copy(src_hbm.at[wid], src_v)
    pltpu.sync_copy(dst_hbm.at[wid], dst_v)
    pltpu.sync_copy(ew_hbm.at[wid], ew_v)

    def fire_gather(j, b):
        pltpu.async_copy(xw_hbm.at[src_v.at[j]], abufs[b], gsems[b])

    def wait_gather(j, b):
        pltpu.make_async_copy(xw_hbm.at[src_v.at[j]], abufs[b], gsems[b]).wait()

    def fire_scatter(j, b):
        pltpu.async_copy(sbufs[b], h_sh.at[dst_v.at[j]], ssems[b], add=True)

    def wait_scatter(j, b):
        pltpu.make_async_copy(sbufs[b], h_sh.at[dst_v.at[j]], ssems[b]).wait()

    def scale(j, b):
        ab = abufs[b]
        sb = sbufs[b]
        for g in range(CHUNK // 16):
            sl = pl.ds(g * 16, 16)
            iv = src_v[j, sl]
            wv = ew_v[j, sl]
            dv = plsc.load_gather(dinv_v, [iv])
            sv = wv * dv
            for l in range(16):
                e = g * 16 + l
                s_ = sv[l]
                for dd in range(4):
                    sl2 = pl.ds(dd * 16, 16)
                    sb[e, sl2] = ab[e, sl2] * s_

    fire_gather(0, 0)
    fire_gather(1, 1)

    @pl.loop(0, NCHUNK // 2)
    def _(jj):
        for b in range(2):
            j = 2 * jj + b
            wait_gather(j, b)

            @pl.when(jj > 0)
            def _():
                wait_scatter(j - 2, b)

            scale(j, b)
            fire_scatter(j, b)

            @pl.when(j + 2 < NCHUNK)
            def _():
                fire_gather(j + 2, b)

    jl = NCHUNK - 1
    wait_gather(jl, 0)
    wait_scatter(jl - 2, 0)
    scale(jl, 0)
    fire_scatter(jl, 0)
    wait_scatter(jl - 1, 1)
    wait_scatter(jl, 0)
    plsc.subcore_barrier()
    pltpu.sync_copy(h_sh.at[pl.ds(s * RPT, RPT)], eh_out.at[c, pl.ds(s * RPT, RPT)])


_msg_kernel = pl.kernel(
    _msg_body,
    out_type=(
        jax.ShapeDtypeStruct((NC, NPAD, D_OUT), jnp.float32),
        jax.ShapeDtypeStruct((NPAD,), jnp.float32),
    ),
    mesh=_MESH,
    compiler_params=pltpu.CompilerParams(needs_layout_passes=False,
                                         use_tc_tiling_on_sc=False),
    scratch_types=[
        pltpu.VMEM_SHARED((NPAD, D_OUT), jnp.float32),
        pltpu.VMEM_SHARED((NPAD,), jnp.float32),
        pltpu.VMEM((NPAD,), jnp.float32),
        pltpu.VMEM((NPAD,), jnp.float32),
        pltpu.VMEM((NCHUNK, CHUNK), jnp.int32),
        pltpu.VMEM((NCHUNK, CHUNK), jnp.int32),
        pltpu.VMEM((NCHUNK, CHUNK), jnp.float32),
        pltpu.VMEM((RPT,), jnp.float32),
        pltpu.VMEM((CHUNK, D_OUT), jnp.float32),
        pltpu.VMEM((CHUNK, D_OUT), jnp.float32),
        pltpu.VMEM((CHUNK, D_OUT), jnp.float32),
        pltpu.VMEM((CHUNK, D_OUT), jnp.float32),
        pltpu.SemaphoreType.DMA,
        pltpu.SemaphoreType.DMA,
        pltpu.SemaphoreType.DMA,
        pltpu.SemaphoreType.DMA,
    ],
)


def _mm_body(x_ref, w_ref, o_ref):
    o_ref[...] = jnp.dot(x_ref[...], w_ref[...], preferred_element_type=jnp.float32)


_mm_kernel = pl.pallas_call(
    _mm_body,
    out_shape=jax.ShapeDtypeStruct((N, D_OUT), jnp.float32),
)


def _fin_body(eh_ref, xw_ref, dinv_ref, bat_ref, bc_ref, wl_ref, bl_ref,
              o_ref, sums, counts):
    i = pl.program_id(0)

    @pl.when(i == 0)
    def _():
        sums[...] = jnp.zeros_like(sums)
        counts[...] = jnp.zeros_like(counts)

    dv = dinv_ref[...]
    t = dv * (eh_ref[0] + eh_ref[1] + dv * xw_ref[...]) + bc_ref[...]
    t = jnp.maximum(t, 0.0)
    oh = (bat_ref[...] == lax.broadcasted_iota(jnp.int32, (BLK, G), 1))
    oh = oh.astype(jnp.float32)
    sums[...] += lax.dot_general(oh, t, (((0,), (0,)), ((), ())),
                                 preferred_element_type=jnp.float32)
    counts[0, :] += jnp.sum(oh, axis=0)

    @pl.when(i == NBLK - 1)
    def _():
        pooled = sums[...] / jnp.maximum(counts[0, :], 1.0)[:, None]
        logits = jnp.dot(pooled, wl_ref[...], preferred_element_type=jnp.float32)
        o_ref[...] = jax.nn.sigmoid(logits + bl_ref[...])


_fin_kernel = pl.pallas_call(
    _fin_body,
    grid=(NBLK,),
    in_specs=[
        pl.BlockSpec((NC, BLK, D_OUT), lambda i: (0, i, 0)),
        pl.BlockSpec((BLK, D_OUT), lambda i: (i, 0)),
        pl.BlockSpec((BLK, 1), lambda i: (i, 0)),
        pl.BlockSpec((BLK, 1), lambda i: (i, 0)),
        pl.BlockSpec((1, D_OUT), lambda i: (0, 0)),
        pl.BlockSpec((D_OUT, 1), lambda i: (0, 0)),
        pl.BlockSpec((1, 1), lambda i: (0, 0)),
    ],
    out_specs=pl.BlockSpec((G, 1), lambda i: (0, 0)),
    out_shape=jax.ShapeDtypeStruct((G, 1), jnp.float32),
    scratch_shapes=[
        pltpu.VMEM((G, D_OUT), jnp.float32),
        pltpu.VMEM((8, D_OUT), jnp.float32),
    ],
)


def kernel(x, edge_index, edge_attr, batch, W_conv, b_conv, W_lin, b_lin):
    src = edge_index[0].reshape(NW, NCHUNK, CHUNK)
    dst = edge_index[1].reshape(NW, NCHUNK, CHUNK)
    ew = edge_attr.reshape(NW, NCHUNK, CHUNK)

    xw = _mm_kernel(x, W_conv)
    eh, dinv = _msg_kernel(xw, src, dst, ew)

    out = _fin_kernel(
        eh, xw, dinv.reshape(NPAD, 1)[:N], batch.reshape(N, 1),
        b_conv.reshape(1, D_OUT), W_lin, b_lin.reshape(1, 1),
    )
    return out


# speedup vs baseline: 44.9113x; 1.0174x over previous
import jax
import jax.numpy as jnp
import numpy as np
from jax import lax
from jax.experimental import pallas as pl
from jax.experimental.pallas import tpu as pltpu
from jax.experimental.pallas import tpu_sc as plsc

N = 10000
E = 320000
D_IN = 128
D_OUT = 64
G = 64

NC = 2
NS = 16
NW = NC * NS
EPW = E // NW
CHUNK = 80
NCHUNK = EPW // CHUNK
NPAD = 10240
RPT = NPAD // NS
BLK = 1000
NBLK = N // BLK

_MESH = plsc.VectorSubcoreMesh(core_axis_name="c", subcore_axis_name="s")


def _msg_body(xw_hbm, src_hbm, dst_hbm, ew_hbm, eh_out, dinv_out,
              h_sh, deg_sh, dinv_v, dga, src_v, dst_v, ew_v, zb,
              ab0, ab1, sb0, sb1, gs0, gs1, ss0, ss1):
    abufs = (ab0, ab1)
    sbufs = (sb0, sb1)
    gsems = (gs0, gs1)
    ssems = (ss0, ss1)
    c = lax.axis_index("c")
    s = lax.axis_index("s")
    wid = c * NS + s

    z = jnp.zeros((16,), jnp.float32)

    @pl.loop(0, RPT // 16)
    def _(i):
        zb[pl.ds(i * 16, 16)] = z

    pltpu.sync_copy(zb, deg_sh.at[pl.ds(s * RPT, RPT)])

    @pl.loop(0, CHUNK)
    def _(r):
        for dd in range(4):
            sb0[r, pl.ds(dd * 16, 16)] = z

    @pl.loop(0, RPT // CHUNK)
    def _(k):
        pltpu.sync_copy(sb0, h_sh.at[pl.ds(s * RPT + k * CHUNK, CHUNK)])

    plsc.subcore_barrier()

    for q in range(2):
        w2 = 2 * s + q
        pltpu.sync_copy(dst_hbm.at[w2], dst_v)
        pltpu.sync_copy(ew_hbm.at[w2], ew_v)

        @pl.loop(0, NCHUNK // 25)
        def _(blk):
            @pl.loop(0, 25)
            def _(i):
                j = blk * 25 + i
                pltpu.async_copy(ew_v.at[j], deg_sh.at[dst_v.at[j]], ss0,
                                 add=True)

            @pl.loop(0, 25)
            def _(i):
                j = blk * 25 + i
                pltpu.make_async_copy(ew_v.at[j], deg_sh.at[dst_v.at[j]],
                                      ss0).wait()

    plsc.subcore_barrier()

    pltpu.sync_copy(deg_sh, dga)

    @pl.loop(0, NPAD // 16)
    def _(i):
        sl = pl.ds(i * 16, 16)
        d = dga[sl] + 1.0
        di = lax.bitcast_convert_type(d, jnp.int32)
        yi = jnp.int32(0x5F3759DF) - lax.shift_right_logical(di, 1)
        y = lax.bitcast_convert_type(yi, jnp.float32)
        y = y * (1.5 - 0.5 * d * y * y)
        y = y * (1.5 - 0.5 * d * y * y)
        y = y * (1.5 - 0.5 * d * y * y)
        dinv_v[sl] = y

    @pl.when(c == 0)
    def _():
        pltpu.sync_copy(dinv_v.at[pl.ds(s * RPT, RPT)],
                        dinv_out.at[pl.ds(s * RPT, RPT)])

    pltpu.sync_copy(src_hbm.at[wid], src_v)
    pltpu.sync_copy(dst_hbm.at[wid], dst_v)
    pltpu.sync_copy(ew_hbm.at[wid], ew_v)

    def fire_gather(j, b):
        pltpu.async_copy(xw_hbm.at[src_v.at[j]], abufs[b], gsems[b])

    def wait_gather(j, b):
        pltpu.make_async_copy(xw_hbm.at[src_v.at[j]], abufs[b], gsems[b]).wait()

    def fire_scatter(j, b):
        pltpu.async_copy(sbufs[b], h_sh.at[dst_v.at[j]], ssems[b], add=True)

    def wait_scatter(j, b):
        pltpu.make_async_copy(sbufs[b], h_sh.at[dst_v.at[j]], ssems[b]).wait()

    def scale(j, b):
        ab = abufs[b]
        sb = sbufs[b]
        for g in range(CHUNK // 16):
            sl = pl.ds(g * 16, 16)
            iv = src_v[j, sl]
            wv = ew_v[j, sl]
            dv = plsc.load_gather(dinv_v, [iv])
            sv = wv * dv
            for l in range(16):
                e = g * 16 + l
                s_ = sv[l]
                for g32 in (0, 32):
                    av = plsc.bitcast(ab[e, pl.ds(g32, 32)], jnp.int32)
                    lo = plsc.bitcast(av << 16, jnp.float32)
                    hi = plsc.bitcast(av & jnp.int32(-65536), jnp.float32)
                    sb[e, pl.ds(g32, 16)] = lo * s_
                    sb[e, pl.ds(g32 + 16, 16)] = hi * s_

    fire_gather(0, 0)
    fire_gather(1, 1)

    @pl.loop(0, NCHUNK // 2)
    def _(jj):
        for b in range(2):
            j = 2 * jj + b
            wait_gather(j, b)

            @pl.when(jj > 0)
            def _():
                wait_scatter(j - 2, b)

            scale(j, b)
            fire_scatter(j, b)

            @pl.when(j + 2 < NCHUNK)
            def _():
                fire_gather(j + 2, b)

    jl = NCHUNK - 1
    wait_gather(jl, 0)
    wait_scatter(jl - 2, 0)
    scale(jl, 0)
    fire_scatter(jl, 0)
    wait_scatter(jl - 1, 1)
    wait_scatter(jl, 0)
    plsc.subcore_barrier()
    pltpu.sync_copy(h_sh.at[pl.ds(s * RPT, RPT)], eh_out.at[c, pl.ds(s * RPT, RPT)])


_msg_kernel = pl.kernel(
    _msg_body,
    out_type=(
        jax.ShapeDtypeStruct((NC, NPAD, D_OUT), jnp.float32),
        jax.ShapeDtypeStruct((NPAD,), jnp.float32),
    ),
    mesh=_MESH,
    compiler_params=pltpu.CompilerParams(needs_layout_passes=False,
                                         use_tc_tiling_on_sc=False),
    scratch_types=[
        pltpu.VMEM_SHARED((NPAD, D_OUT), jnp.float32),
        pltpu.VMEM_SHARED((NPAD,), jnp.float32),
        pltpu.VMEM((NPAD,), jnp.float32),
        pltpu.VMEM((NPAD,), jnp.float32),
        pltpu.VMEM((NCHUNK, CHUNK), jnp.int32),
        pltpu.VMEM((NCHUNK, CHUNK), jnp.int32),
        pltpu.VMEM((NCHUNK, CHUNK), jnp.float32),
        pltpu.VMEM((RPT,), jnp.float32),
        pltpu.VMEM((CHUNK, D_OUT), jnp.bfloat16),
        pltpu.VMEM((CHUNK, D_OUT), jnp.bfloat16),
        pltpu.VMEM((CHUNK, D_OUT), jnp.float32),
        pltpu.VMEM((CHUNK, D_OUT), jnp.float32),
        pltpu.SemaphoreType.DMA,
        pltpu.SemaphoreType.DMA,
        pltpu.SemaphoreType.DMA,
        pltpu.SemaphoreType.DMA,
    ],
)


def _mm_body(x_ref, w_ref, wp_ref, o_ref, op_ref):
    xv = x_ref[...]
    o_ref[...] = jnp.dot(xv, w_ref[...], preferred_element_type=jnp.float32)
    op_ref[...] = jnp.dot(xv, wp_ref[...],
                          preferred_element_type=jnp.float32).astype(jnp.bfloat16)


_mm_kernel = pl.pallas_call(
    _mm_body,
    out_shape=(
        jax.ShapeDtypeStruct((N, D_OUT), jnp.float32),
        jax.ShapeDtypeStruct((N, D_OUT), jnp.bfloat16),
    ),
)

_PERM = np.empty((D_OUT,), np.int32)
for _g in (0, 32):
    for _l in range(16):
        _PERM[_g + 2 * _l] = _g + _l
        _PERM[_g + 2 * _l + 1] = _g + 16 + _l


def _fin_body(eh_ref, xw_ref, dinv_ref, bat_ref, bc_ref, wl_ref, bl_ref,
              o_ref, sums, counts):
    i = pl.program_id(0)

    @pl.when(i == 0)
    def _():
        sums[...] = jnp.zeros_like(sums)
        counts[...] = jnp.zeros_like(counts)

    dv = dinv_ref[...]
    t = dv * (eh_ref[0] + eh_ref[1] + dv * xw_ref[...]) + bc_ref[...]
    t = jnp.maximum(t, 0.0)
    oh = (bat_ref[...] == lax.broadcasted_iota(jnp.int32, (BLK, G), 1))
    oh = oh.astype(jnp.float32)
    sums[...] += lax.dot_general(oh, t, (((0,), (0,)), ((), ())),
                                 preferred_element_type=jnp.float32)
    counts[0, :] += jnp.sum(oh, axis=0)

    @pl.when(i == NBLK - 1)
    def _():
        pooled = sums[...] / jnp.maximum(counts[0, :], 1.0)[:, None]
        logits = jnp.dot(pooled, wl_ref[...], preferred_element_type=jnp.float32)
        o_ref[...] = jax.nn.sigmoid(logits + bl_ref[...])


_fin_kernel = pl.pallas_call(
    _fin_body,
    grid=(NBLK,),
    in_specs=[
        pl.BlockSpec((NC, BLK, D_OUT), lambda i: (0, i, 0)),
        pl.BlockSpec((BLK, D_OUT), lambda i: (i, 0)),
        pl.BlockSpec((BLK, 1), lambda i: (i, 0)),
        pl.BlockSpec((BLK, 1), lambda i: (i, 0)),
        pl.BlockSpec((1, D_OUT), lambda i: (0, 0)),
        pl.BlockSpec((D_OUT, 1), lambda i: (0, 0)),
        pl.BlockSpec((1, 1), lambda i: (0, 0)),
    ],
    out_specs=pl.BlockSpec((G, 1), lambda i: (0, 0)),
    out_shape=jax.ShapeDtypeStruct((G, 1), jnp.float32),
    scratch_shapes=[
        pltpu.VMEM((G, D_OUT), jnp.float32),
        pltpu.VMEM((8, D_OUT), jnp.float32),
    ],
)


def kernel(x, edge_index, edge_attr, batch, W_conv, b_conv, W_lin, b_lin):
    src = edge_index[0].reshape(NW, NCHUNK, CHUNK)
    dst = edge_index[1].reshape(NW, NCHUNK, CHUNK)
    ew = edge_attr.reshape(NW, NCHUNK, CHUNK)

    xw, xw_bf = _mm_kernel(x, W_conv, W_conv[:, _PERM])
    eh, dinv = _msg_kernel(xw_bf, src, dst, ew)

    out = _fin_kernel(
        eh, xw, dinv.reshape(NPAD, 1)[:N], batch.reshape(N, 1),
        b_conv.reshape(1, D_OUT), W_lin, b_lin.reshape(1, 1),
    )
    return out


# speedup vs baseline: 46.8710x; 1.0436x over previous
import jax
import jax.numpy as jnp
import numpy as np
from jax import lax
from jax.experimental import pallas as pl
from jax.experimental.pallas import tpu as pltpu
from jax.experimental.pallas import tpu_sc as plsc

N = 10000
E = 320000
D_IN = 128
D_OUT = 64
G = 64

NC = 2
NS = 16
NW = NC * NS
EPW = E // NW
CHUNK = 80
NCHUNK = EPW // CHUNK
NPAD = 10240
RPT = NPAD // NS
BLK = 1000
NBLK = N // BLK

_MESH = plsc.VectorSubcoreMesh(core_axis_name="c", subcore_axis_name="s")


def _msg_body(xw_hbm, src_hbm, dst_hbm, ew_hbm, eh_out, dinv_out,
              h_sh, deg_sh, dinv_v, dga, src_v, dst_v, ew_v, zb,
              ab0, ab1, sb0, sb1, gs0, gs1, ss0, ss1):
    abufs = (ab0, ab1)
    sbufs = (sb0, sb1)
    gsems = (gs0, gs1)
    ssems = (ss0, ss1)
    c = lax.axis_index("c")
    s = lax.axis_index("s")
    wid = c * NS + s

    z = jnp.zeros((16,), jnp.float32)

    @pl.loop(0, RPT // 16)
    def _(i):
        zb[pl.ds(i * 16, 16)] = z

    pltpu.sync_copy(zb, deg_sh.at[pl.ds(s * RPT, RPT)])

    zb16 = jnp.zeros((32,), jnp.bfloat16)

    @pl.loop(0, CHUNK)
    def _(r):
        for dd in range(2):
            sb0[r, pl.ds(dd * 32, 32)] = zb16

    @pl.loop(0, RPT // CHUNK)
    def _(k):
        pltpu.sync_copy(sb0, h_sh.at[pl.ds(s * RPT + k * CHUNK, CHUNK)])

    plsc.subcore_barrier()

    for q in range(2):
        w2 = 2 * s + q
        pltpu.sync_copy(dst_hbm.at[w2], dst_v)
        pltpu.sync_copy(ew_hbm.at[w2], ew_v)

        @pl.loop(0, NCHUNK // 25)
        def _(blk):
            @pl.loop(0, 25)
            def _(i):
                j = blk * 25 + i
                pltpu.async_copy(ew_v.at[j], deg_sh.at[dst_v.at[j]], ss0,
                                 add=True)

            @pl.loop(0, 25)
            def _(i):
                j = blk * 25 + i
                pltpu.make_async_copy(ew_v.at[j], deg_sh.at[dst_v.at[j]],
                                      ss0).wait()

    plsc.subcore_barrier()

    pltpu.sync_copy(deg_sh, dga)

    @pl.loop(0, NPAD // 16)
    def _(i):
        sl = pl.ds(i * 16, 16)
        d = dga[sl] + 1.0
        di = lax.bitcast_convert_type(d, jnp.int32)
        yi = jnp.int32(0x5F3759DF) - lax.shift_right_logical(di, 1)
        y = lax.bitcast_convert_type(yi, jnp.float32)
        y = y * (1.5 - 0.5 * d * y * y)
        y = y * (1.5 - 0.5 * d * y * y)
        y = y * (1.5 - 0.5 * d * y * y)
        dinv_v[sl] = y

    @pl.when(c == 0)
    def _():
        pltpu.sync_copy(dinv_v.at[pl.ds(s * RPT, RPT)],
                        dinv_out.at[pl.ds(s * RPT, RPT)])

    pltpu.sync_copy(src_hbm.at[wid], src_v)
    pltpu.sync_copy(dst_hbm.at[wid], dst_v)
    pltpu.sync_copy(ew_hbm.at[wid], ew_v)

    def fire_gather(j, b):
        pltpu.async_copy(xw_hbm.at[src_v.at[j]], abufs[b], gsems[b])

    def wait_gather(j, b):
        pltpu.make_async_copy(xw_hbm.at[src_v.at[j]], abufs[b], gsems[b]).wait()

    def fire_scatter(j, b):
        pltpu.async_copy(sbufs[b], h_sh.at[dst_v.at[j]], ssems[b], add=True)

    def wait_scatter(j, b):
        pltpu.make_async_copy(sbufs[b], h_sh.at[dst_v.at[j]], ssems[b]).wait()

    def scale(j, b):
        ab = abufs[b]
        sb = sbufs[b]
        for g in range(CHUNK // 16):
            sl = pl.ds(g * 16, 16)
            iv = src_v[j, sl]
            wv = ew_v[j, sl]
            dv = plsc.load_gather(dinv_v, [iv])
            sv = wv * dv
            for l in range(16):
                e = g * 16 + l
                s_ = sv[l]
                for g32 in (0, 32):
                    av = plsc.bitcast(ab[e, pl.ds(g32, 32)], jnp.int32)
                    lo = plsc.bitcast(av << 16, jnp.float32)
                    hi = plsc.bitcast(av & jnp.int32(-65536), jnp.float32)
                    sb[e, pl.ds(g32, 32)] = plsc.pack(
                        lo * s_, hi * s_, format=plsc.PackFormat.INTERLEAVED)

    fire_gather(0, 0)
    fire_gather(1, 1)

    @pl.loop(0, NCHUNK // 2)
    def _(jj):
        for b in range(2):
            j = 2 * jj + b
            wait_gather(j, b)

            @pl.when(jj > 0)
            def _():
                wait_scatter(j - 2, b)

            scale(j, b)
            fire_scatter(j, b)

            @pl.when(j + 2 < NCHUNK)
            def _():
                fire_gather(j + 2, b)

    jl = NCHUNK - 1
    wait_gather(jl, 0)
    wait_scatter(jl - 2, 0)
    scale(jl, 0)
    fire_scatter(jl, 0)
    wait_scatter(jl - 1, 1)
    wait_scatter(jl, 0)
    plsc.subcore_barrier()
    pltpu.sync_copy(h_sh.at[pl.ds(s * RPT, RPT)], eh_out.at[c, pl.ds(s * RPT, RPT)])


_msg_kernel = pl.kernel(
    _msg_body,
    out_type=(
        jax.ShapeDtypeStruct((NC, NPAD, D_OUT), jnp.bfloat16),
        jax.ShapeDtypeStruct((NPAD,), jnp.float32),
    ),
    mesh=_MESH,
    compiler_params=pltpu.CompilerParams(needs_layout_passes=False,
                                         use_tc_tiling_on_sc=False),
    scratch_types=[
        pltpu.VMEM_SHARED((NPAD, D_OUT), jnp.bfloat16),
        pltpu.VMEM_SHARED((NPAD,), jnp.float32),
        pltpu.VMEM((NPAD,), jnp.float32),
        pltpu.VMEM((NPAD,), jnp.float32),
        pltpu.VMEM((NCHUNK, CHUNK), jnp.int32),
        pltpu.VMEM((NCHUNK, CHUNK), jnp.int32),
        pltpu.VMEM((NCHUNK, CHUNK), jnp.float32),
        pltpu.VMEM((RPT,), jnp.float32),
        pltpu.VMEM((CHUNK, D_OUT), jnp.bfloat16),
        pltpu.VMEM((CHUNK, D_OUT), jnp.bfloat16),
        pltpu.VMEM((CHUNK, D_OUT), jnp.bfloat16),
        pltpu.VMEM((CHUNK, D_OUT), jnp.bfloat16),
        pltpu.SemaphoreType.DMA,
        pltpu.SemaphoreType.DMA,
        pltpu.SemaphoreType.DMA,
        pltpu.SemaphoreType.DMA,
    ],
)


def _mm_body(x_ref, w_ref, o_ref, op_ref):
    xv = jnp.dot(x_ref[...], w_ref[...], preferred_element_type=jnp.float32)
    o_ref[...] = xv
    op_ref[...] = xv.astype(jnp.bfloat16)


_PERM = np.empty((D_OUT,), np.int32)
for _g in (0, 32):
    for _l in range(16):
        _PERM[_g + 2 * _l] = _g + _l
        _PERM[_g + 2 * _l + 1] = _g + 16 + _l


_mm_kernel = pl.pallas_call(
    _mm_body,
    out_shape=(
        jax.ShapeDtypeStruct((N, D_OUT), jnp.float32),
        jax.ShapeDtypeStruct((N, D_OUT), jnp.bfloat16),
    ),
)


def _fin_body(eh_ref, xw_ref, dinv_ref, bat_ref, bc_ref, wl_ref, bl_ref,
              o_ref, sums, counts):
    i = pl.program_id(0)

    @pl.when(i == 0)
    def _():
        sums[...] = jnp.zeros_like(sums)
        counts[...] = jnp.zeros_like(counts)

    dv = dinv_ref[...]
    ehs = eh_ref[0].astype(jnp.float32) + eh_ref[1].astype(jnp.float32)
    t = dv * (ehs + dv * xw_ref[...]) + bc_ref[...]
    t = jnp.maximum(t, 0.0)
    oh = (bat_ref[...] == lax.broadcasted_iota(jnp.int32, (BLK, G), 1))
    oh = oh.astype(jnp.float32)
    sums[...] += lax.dot_general(oh, t, (((0,), (0,)), ((), ())),
                                 preferred_element_type=jnp.float32)
    counts[0, :] += jnp.sum(oh, axis=0)

    @pl.when(i == NBLK - 1)
    def _():
        pooled = sums[...] / jnp.maximum(counts[0, :], 1.0)[:, None]
        logits = jnp.dot(pooled, wl_ref[...], preferred_element_type=jnp.float32)
        o_ref[...] = jax.nn.sigmoid(logits + bl_ref[...])


_fin_kernel = pl.pallas_call(
    _fin_body,
    grid=(NBLK,),
    in_specs=[
        pl.BlockSpec((NC, BLK, D_OUT), lambda i: (0, i, 0)),
        pl.BlockSpec((BLK, D_OUT), lambda i: (i, 0)),
        pl.BlockSpec((BLK, 1), lambda i: (i, 0)),
        pl.BlockSpec((BLK, 1), lambda i: (i, 0)),
        pl.BlockSpec((1, D_OUT), lambda i: (0, 0)),
        pl.BlockSpec((D_OUT, 1), lambda i: (0, 0)),
        pl.BlockSpec((1, 1), lambda i: (0, 0)),
    ],
    out_specs=pl.BlockSpec((G, 1), lambda i: (0, 0)),
    out_shape=jax.ShapeDtypeStruct((G, 1), jnp.float32),
    scratch_shapes=[
        pltpu.VMEM((G, D_OUT), jnp.float32),
        pltpu.VMEM((8, D_OUT), jnp.float32),
    ],
)


def kernel(x, edge_index, edge_attr, batch, W_conv, b_conv, W_lin, b_lin):
    src = edge_index[0].reshape(NW, NCHUNK, CHUNK)
    dst = edge_index[1].reshape(NW, NCHUNK, CHUNK)
    ew = edge_attr.reshape(NW, NCHUNK, CHUNK)

    xw, xw_bf = _mm_kernel(x, W_conv[:, _PERM])
    eh, dinv = _msg_kernel(xw_bf, src, dst, ew)

    out = _fin_kernel(
        eh, xw, dinv.reshape(NPAD, 1)[:N], batch.reshape(N, 1),
        b_conv[_PERM].reshape(1, D_OUT), W_lin[_PERM, :], b_lin.reshape(1, 1),
    )
    return out


# speedup vs baseline: 47.8779x; 1.0215x over previous
import jax
import jax.numpy as jnp
import numpy as np
from jax import lax
from jax.experimental import pallas as pl
from jax.experimental.pallas import tpu as pltpu
from jax.experimental.pallas import tpu_sc as plsc

N = 10000
E = 320000
D_IN = 128
D_OUT = 64
G = 64

NC = 2
NS = 16
NW = NC * NS
EPW = E // NW
CHUNK = 80
NCHUNK = EPW // CHUNK
NPAD = 10240
RPT = NPAD // NS
BLK = 1000
NBLK = N // BLK

_MESH = plsc.VectorSubcoreMesh(core_axis_name="c", subcore_axis_name="s")


def _msg_body(xw_hbm, src_hbm, dst_hbm, ew_hbm, eh_out, dinv_out,
              h_sh, deg_sh, dinv_v, dga, src_v, dst_v, ew_v, zb,
              ab0, ab1, sb0, sb1, gs0, gs1, ss0, ss1):
    abufs = (ab0, ab1)
    sbufs = (sb0, sb1)
    gsems = (gs0, gs1)
    ssems = (ss0, ss1)
    c = lax.axis_index("c")
    s = lax.axis_index("s")
    wid = c * NS + s

    z = jnp.zeros((16,), jnp.float32)

    @pl.loop(0, RPT // 16)
    def _(i):
        zb[pl.ds(i * 16, 16)] = z

    pltpu.sync_copy(zb, deg_sh.at[pl.ds(s * RPT, RPT)])

    zb16 = jnp.zeros((32,), jnp.bfloat16)

    @pl.loop(0, CHUNK)
    def _(r):
        for dd in range(2):
            sb0[r, pl.ds(dd * 32, 32)] = zb16

    @pl.loop(0, RPT // CHUNK)
    def _(k):
        pltpu.sync_copy(sb0, h_sh.at[pl.ds(s * RPT + k * CHUNK, CHUNK)])

    plsc.subcore_barrier()

    for q in range(2):
        w2 = 2 * s + q
        pltpu.sync_copy(dst_hbm.at[w2], dst_v)
        pltpu.sync_copy(ew_hbm.at[w2], ew_v)

        @pl.loop(0, NCHUNK // 25)
        def _(blk):
            @pl.loop(0, 25)
            def _(i):
                j = blk * 25 + i
                pltpu.async_copy(ew_v.at[j], deg_sh.at[dst_v.at[j]], ss0,
                                 add=True)

            @pl.loop(0, 25)
            def _(i):
                j = blk * 25 + i
                pltpu.make_async_copy(ew_v.at[j], deg_sh.at[dst_v.at[j]],
                                      ss0).wait()

    plsc.subcore_barrier()

    pltpu.sync_copy(deg_sh, dga)

    @pl.loop(0, NPAD // 16)
    def _(i):
        sl = pl.ds(i * 16, 16)
        d = dga[sl] + 1.0
        di = lax.bitcast_convert_type(d, jnp.int32)
        yi = jnp.int32(0x5F3759DF) - lax.shift_right_logical(di, 1)
        y = lax.bitcast_convert_type(yi, jnp.float32)
        y = y * (1.5 - 0.5 * d * y * y)
        y = y * (1.5 - 0.5 * d * y * y)
        y = y * (1.5 - 0.5 * d * y * y)
        dinv_v[sl] = y

    @pl.when(c == 0)
    def _():
        pltpu.sync_copy(dinv_v.at[pl.ds(s * RPT, RPT)],
                        dinv_out.at[pl.ds(s * RPT, RPT)])

    pltpu.sync_copy(src_hbm.at[wid], src_v)
    pltpu.sync_copy(dst_hbm.at[wid], dst_v)
    pltpu.sync_copy(ew_hbm.at[wid], ew_v)

    def fire_gather(j, b):
        pltpu.async_copy(xw_hbm.at[src_v.at[j]], abufs[b], gsems[b])

    def wait_gather(j, b):
        pltpu.make_async_copy(xw_hbm.at[src_v.at[j]], abufs[b], gsems[b]).wait()

    def fire_scatter(j, b):
        pltpu.async_copy(sbufs[b], h_sh.at[dst_v.at[j]], ssems[b], add=True)

    def wait_scatter(j, b):
        pltpu.make_async_copy(sbufs[b], h_sh.at[dst_v.at[j]], ssems[b]).wait()

    def scale(j, b):
        ab = abufs[b]
        sb = sbufs[b]
        for g in range(CHUNK // 16):
            sl = pl.ds(g * 16, 16)
            iv = src_v[j, sl]
            wv = ew_v[j, sl]
            dv = plsc.load_gather(dinv_v, [iv])
            sv = wv * dv
            pp = plsc.bitcast(
                plsc.pack(sv, sv, format=plsc.PackFormat.INTERLEAVED),
                jnp.int32)
            for l in range(16):
                e = g * 16 + l
                s_b = plsc.bitcast(jnp.full((16,), pp[l], jnp.int32),
                                   jnp.bfloat16)
                for g32 in (0, 32):
                    sl2 = pl.ds(g32, 32)
                    sb[e, sl2] = ab[e, sl2] * s_b

    fire_gather(0, 0)
    fire_gather(1, 1)

    @pl.loop(0, NCHUNK // 2)
    def _(jj):
        for b in range(2):
            j = 2 * jj + b
            wait_gather(j, b)

            @pl.when(jj > 0)
            def _():
                wait_scatter(j - 2, b)

            scale(j, b)
            fire_scatter(j, b)

            @pl.when(j + 2 < NCHUNK)
            def _():
                fire_gather(j + 2, b)

    jl = NCHUNK - 1
    wait_gather(jl, 0)
    wait_scatter(jl - 2, 0)
    scale(jl, 0)
    fire_scatter(jl, 0)
    wait_scatter(jl - 1, 1)
    wait_scatter(jl, 0)
    plsc.subcore_barrier()
    pltpu.sync_copy(h_sh.at[pl.ds(s * RPT, RPT)], eh_out.at[c, pl.ds(s * RPT, RPT)])


_msg_kernel = pl.kernel(
    _msg_body,
    out_type=(
        jax.ShapeDtypeStruct((NC, NPAD, D_OUT), jnp.bfloat16),
        jax.ShapeDtypeStruct((NPAD,), jnp.float32),
    ),
    mesh=_MESH,
    compiler_params=pltpu.CompilerParams(needs_layout_passes=False,
                                         use_tc_tiling_on_sc=False),
    scratch_types=[
        pltpu.VMEM_SHARED((NPAD, D_OUT), jnp.bfloat16),
        pltpu.VMEM_SHARED((NPAD,), jnp.float32),
        pltpu.VMEM((NPAD,), jnp.float32),
        pltpu.VMEM((NPAD,), jnp.float32),
        pltpu.VMEM((NCHUNK, CHUNK), jnp.int32),
        pltpu.VMEM((NCHUNK, CHUNK), jnp.int32),
        pltpu.VMEM((NCHUNK, CHUNK), jnp.float32),
        pltpu.VMEM((RPT,), jnp.float32),
        pltpu.VMEM((CHUNK, D_OUT), jnp.bfloat16),
        pltpu.VMEM((CHUNK, D_OUT), jnp.bfloat16),
        pltpu.VMEM((CHUNK, D_OUT), jnp.bfloat16),
        pltpu.VMEM((CHUNK, D_OUT), jnp.bfloat16),
        pltpu.SemaphoreType.DMA,
        pltpu.SemaphoreType.DMA,
        pltpu.SemaphoreType.DMA,
        pltpu.SemaphoreType.DMA,
    ],
)


def _mm_body(x_ref, w_ref, o_ref, op_ref):
    xv = jnp.dot(x_ref[...], w_ref[...], preferred_element_type=jnp.float32)
    o_ref[...] = xv
    op_ref[...] = xv.astype(jnp.bfloat16)


_PERM = np.empty((D_OUT,), np.int32)
for _g in (0, 32):
    for _l in range(16):
        _PERM[_g + 2 * _l] = _g + _l
        _PERM[_g + 2 * _l + 1] = _g + 16 + _l


_mm_kernel = pl.pallas_call(
    _mm_body,
    out_shape=(
        jax.ShapeDtypeStruct((N, D_OUT), jnp.float32),
        jax.ShapeDtypeStruct((N, D_OUT), jnp.bfloat16),
    ),
)


def _fin_body(eh_ref, xw_ref, dinv_ref, bat_ref, bc_ref, wl_ref, bl_ref,
              o_ref, sums, counts):
    i = pl.program_id(0)

    @pl.when(i == 0)
    def _():
        sums[...] = jnp.zeros_like(sums)
        counts[...] = jnp.zeros_like(counts)

    dv = dinv_ref[...]
    ehs = eh_ref[0].astype(jnp.float32) + eh_ref[1].astype(jnp.float32)
    t = dv * (ehs + dv * xw_ref[...]) + bc_ref[...]
    t = jnp.maximum(t, 0.0)
    oh = (bat_ref[...] == lax.broadcasted_iota(jnp.int32, (BLK, G), 1))
    oh = oh.astype(jnp.float32)
    sums[...] += lax.dot_general(oh, t, (((0,), (0,)), ((), ())),
                                 preferred_element_type=jnp.float32)
    counts[0, :] += jnp.sum(oh, axis=0)

    @pl.when(i == NBLK - 1)
    def _():
        pooled = sums[...] / jnp.maximum(counts[0, :], 1.0)[:, None]
        logits = jnp.dot(pooled, wl_ref[...], preferred_element_type=jnp.float32)
        o_ref[...] = jax.nn.sigmoid(logits + bl_ref[...])


_fin_kernel = pl.pallas_call(
    _fin_body,
    grid=(NBLK,),
    in_specs=[
        pl.BlockSpec((NC, BLK, D_OUT), lambda i: (0, i, 0)),
        pl.BlockSpec((BLK, D_OUT), lambda i: (i, 0)),
        pl.BlockSpec((BLK, 1), lambda i: (i, 0)),
        pl.BlockSpec((BLK, 1), lambda i: (i, 0)),
        pl.BlockSpec((1, D_OUT), lambda i: (0, 0)),
        pl.BlockSpec((D_OUT, 1), lambda i: (0, 0)),
        pl.BlockSpec((1, 1), lambda i: (0, 0)),
    ],
    out_specs=pl.BlockSpec((G, 1), lambda i: (0, 0)),
    out_shape=jax.ShapeDtypeStruct((G, 1), jnp.float32),
    scratch_shapes=[
        pltpu.VMEM((G, D_OUT), jnp.float32),
        pltpu.VMEM((8, D_OUT), jnp.float32),
    ],
)


def kernel(x, edge_index, edge_attr, batch, W_conv, b_conv, W_lin, b_lin):
    src = edge_index[0].reshape(NW, NCHUNK, CHUNK)
    dst = edge_index[1].reshape(NW, NCHUNK, CHUNK)
    ew = edge_attr.reshape(NW, NCHUNK, CHUNK)

    xw, xw_bf = _mm_kernel(x, W_conv[:, _PERM])
    eh, dinv = _msg_kernel(xw_bf, src, dst, ew)

    out = _fin_kernel(
        eh, xw, dinv.reshape(NPAD, 1)[:N], batch.reshape(N, 1),
        b_conv[_PERM].reshape(1, D_OUT), W_lin[_PERM, :], b_lin.reshape(1, 1),
    )
    return out


# speedup vs baseline: 47.8915x; 1.0003x over previous
import jax
import jax.numpy as jnp
import numpy as np
from jax import lax
from jax.experimental import pallas as pl
from jax.experimental.pallas import tpu as pltpu
from jax.experimental.pallas import tpu_sc as plsc

N = 10000
E = 320000
D_IN = 128
D_OUT = 64
G = 64

NC = 2
NS = 16
NW = NC * NS
EPW = E // NW
CHUNK = 80
NCHUNK = EPW // CHUNK
NPAD = 10240
RPT = NPAD // NS
BLK = 1000
NBLK = N // BLK

_MESH = plsc.VectorSubcoreMesh(core_axis_name="c", subcore_axis_name="s")


def _msg_body(xw_hbm, src_hbm, dst_hbm, ew_hbm, eh_out, dinv_out,
              h_sh, deg_sh, dinv_v, dga, src_v, dst_v, ew_v, dv2, ev2, zb,
              ab0, ab1, sb0, sb1, gs0, gs1, ss0, ss1):
    abufs = (ab0, ab1)
    sbufs = (sb0, sb1)
    gsems = (gs0, gs1)
    ssems = (ss0, ss1)
    c = lax.axis_index("c")
    s = lax.axis_index("s")
    wid = c * NS + s

    z = jnp.zeros((16,), jnp.float32)

    @pl.loop(0, RPT // 16)
    def _(i):
        zb[pl.ds(i * 16, 16)] = z

    pltpu.sync_copy(zb, deg_sh.at[pl.ds(s * RPT, RPT)])

    pltpu.sync_copy(src_hbm.at[wid], src_v)
    pltpu.async_copy(xw_hbm.at[src_v.at[0]], ab0, gs0)
    pltpu.async_copy(xw_hbm.at[src_v.at[1]], ab1, gs1)

    zb16 = jnp.zeros((32,), jnp.bfloat16)

    @pl.loop(0, CHUNK)
    def _(r):
        for dd in range(2):
            sb0[r, pl.ds(dd * 32, 32)] = zb16

    @pl.loop(0, RPT // CHUNK)
    def _(k):
        pltpu.sync_copy(sb0, h_sh.at[pl.ds(s * RPT + k * CHUNK, CHUNK)])

    plsc.subcore_barrier()

    pltpu.sync_copy(dst_hbm.at[2 * s], dst_v)
    pltpu.sync_copy(ew_hbm.at[2 * s], ew_v)
    pltpu.sync_copy(dst_hbm.at[2 * s + 1], dv2)
    pltpu.sync_copy(ew_hbm.at[2 * s + 1], ev2)
    for q in range(2):
        dref = (dst_v, dv2)[q]
        eref = (ew_v, ev2)[q]

        @pl.loop(0, NCHUNK // 25)
        def _(blk):
            @pl.loop(0, 25)
            def _(i):
                j = blk * 25 + i
                pltpu.async_copy(eref.at[j], deg_sh.at[dref.at[j]], ss0,
                                 add=True)

            @pl.loop(0, 25)
            def _(i):
                j = blk * 25 + i
                pltpu.make_async_copy(eref.at[j], deg_sh.at[dref.at[j]],
                                      ss0).wait()

    plsc.subcore_barrier()

    pltpu.sync_copy(deg_sh, dga)

    @pl.loop(0, NPAD // 16)
    def _(i):
        sl = pl.ds(i * 16, 16)
        d = dga[sl] + 1.0
        di = lax.bitcast_convert_type(d, jnp.int32)
        yi = jnp.int32(0x5F3759DF) - lax.shift_right_logical(di, 1)
        y = lax.bitcast_convert_type(yi, jnp.float32)
        y = y * (1.5 - 0.5 * d * y * y)
        y = y * (1.5 - 0.5 * d * y * y)
        y = y * (1.5 - 0.5 * d * y * y)
        dinv_v[sl] = y

    @pl.when(c == 0)
    def _():
        pltpu.sync_copy(dinv_v.at[pl.ds(s * RPT, RPT)],
                        dinv_out.at[pl.ds(s * RPT, RPT)])

    pltpu.sync_copy(dst_hbm.at[wid], dst_v)
    pltpu.sync_copy(ew_hbm.at[wid], ew_v)

    def fire_gather(j, b):
        pltpu.async_copy(xw_hbm.at[src_v.at[j]], abufs[b], gsems[b])

    def wait_gather(j, b):
        pltpu.make_async_copy(xw_hbm.at[src_v.at[j]], abufs[b], gsems[b]).wait()

    def fire_scatter(j, b):
        pltpu.async_copy(sbufs[b], h_sh.at[dst_v.at[j]], ssems[b], add=True)

    def wait_scatter(j, b):
        pltpu.make_async_copy(sbufs[b], h_sh.at[dst_v.at[j]], ssems[b]).wait()

    def scale(j, b):
        ab = abufs[b]
        sb = sbufs[b]
        for g in range(CHUNK // 16):
            sl = pl.ds(g * 16, 16)
            iv = src_v[j, sl]
            wv = ew_v[j, sl]
            dv = plsc.load_gather(dinv_v, [iv])
            sv = wv * dv
            pp = plsc.bitcast(
                plsc.pack(sv, sv, format=plsc.PackFormat.INTERLEAVED),
                jnp.int32)
            for l in range(16):
                e = g * 16 + l
                s_b = plsc.bitcast(jnp.full((16,), pp[l], jnp.int32),
                                   jnp.bfloat16)
                for g32 in (0, 32):
                    sl2 = pl.ds(g32, 32)
                    sb[e, sl2] = ab[e, sl2] * s_b

    @pl.loop(0, NCHUNK // 2)
    def _(jj):
        for b in range(2):
            j = 2 * jj + b
            wait_gather(j, b)

            @pl.when(jj > 0)
            def _():
                wait_scatter(j - 2, b)

            scale(j, b)
            fire_scatter(j, b)

            @pl.when(j + 2 < NCHUNK)
            def _():
                fire_gather(j + 2, b)

    jl = NCHUNK - 1
    wait_gather(jl, 0)
    wait_scatter(jl - 2, 0)
    scale(jl, 0)
    fire_scatter(jl, 0)
    wait_scatter(jl - 1, 1)
    wait_scatter(jl, 0)
    plsc.subcore_barrier()
    pltpu.sync_copy(h_sh.at[pl.ds(s * RPT, RPT)], eh_out.at[c, pl.ds(s * RPT, RPT)])


_msg_kernel = pl.kernel(
    _msg_body,
    out_type=(
        jax.ShapeDtypeStruct((NC, NPAD, D_OUT), jnp.bfloat16),
        jax.ShapeDtypeStruct((NPAD,), jnp.float32),
    ),
    mesh=_MESH,
    compiler_params=pltpu.CompilerParams(needs_layout_passes=False,
                                         use_tc_tiling_on_sc=False),
    scratch_types=[
        pltpu.VMEM_SHARED((NPAD, D_OUT), jnp.bfloat16),
        pltpu.VMEM_SHARED((NPAD,), jnp.float32),
        pltpu.VMEM((NPAD,), jnp.float32),
        pltpu.VMEM((NPAD,), jnp.float32),
        pltpu.VMEM((NCHUNK, CHUNK), jnp.int32),
        pltpu.VMEM((NCHUNK, CHUNK), jnp.int32),
        pltpu.VMEM((NCHUNK, CHUNK), jnp.float32),
        pltpu.VMEM((NCHUNK, CHUNK), jnp.int32),
        pltpu.VMEM((NCHUNK, CHUNK), jnp.float32),
        pltpu.VMEM((RPT,), jnp.float32),
        pltpu.VMEM((CHUNK, D_OUT), jnp.bfloat16),
        pltpu.VMEM((CHUNK, D_OUT), jnp.bfloat16),
        pltpu.VMEM((CHUNK, D_OUT), jnp.bfloat16),
        pltpu.VMEM((CHUNK, D_OUT), jnp.bfloat16),
        pltpu.SemaphoreType.DMA,
        pltpu.SemaphoreType.DMA,
        pltpu.SemaphoreType.DMA,
        pltpu.SemaphoreType.DMA,
    ],
)


def _mm_body(x_ref, w_ref, o_ref, op_ref):
    xv = jnp.dot(x_ref[...], w_ref[...], preferred_element_type=jnp.float32)
    o_ref[...] = xv
    op_ref[...] = xv.astype(jnp.bfloat16)


_PERM = np.empty((D_OUT,), np.int32)
for _g in (0, 32):
    for _l in range(16):
        _PERM[_g + 2 * _l] = _g + _l
        _PERM[_g + 2 * _l + 1] = _g + 16 + _l


_mm_kernel = pl.pallas_call(
    _mm_body,
    out_shape=(
        jax.ShapeDtypeStruct((N, D_OUT), jnp.float32),
        jax.ShapeDtypeStruct((N, D_OUT), jnp.bfloat16),
    ),
)


def _fin_body(eh_ref, xw_ref, dinv_ref, bat_ref, bc_ref, wl_ref, bl_ref,
              o_ref, sums, counts):
    i = pl.program_id(0)

    @pl.when(i == 0)
    def _():
        sums[...] = jnp.zeros_like(sums)
        counts[...] = jnp.zeros_like(counts)

    dv = dinv_ref[...]
    ehs = eh_ref[0].astype(jnp.float32) + eh_ref[1].astype(jnp.float32)
    t = dv * (ehs + dv * xw_ref[...]) + bc_ref[...]
    t = jnp.maximum(t, 0.0)
    oh = (bat_ref[...] == lax.broadcasted_iota(jnp.int32, (BLK, G), 1))
    oh = oh.astype(jnp.float32)
    sums[...] += lax.dot_general(oh, t, (((0,), (0,)), ((), ())),
                                 preferred_element_type=jnp.float32)
    counts[0, :] += jnp.sum(oh, axis=0)

    @pl.when(i == NBLK - 1)
    def _():
        pooled = sums[...] / jnp.maximum(counts[0, :], 1.0)[:, None]
        logits = jnp.dot(pooled, wl_ref[...], preferred_element_type=jnp.float32)
        o_ref[...] = jax.nn.sigmoid(logits + bl_ref[...])


_fin_kernel = pl.pallas_call(
    _fin_body,
    grid=(NBLK,),
    in_specs=[
        pl.BlockSpec((NC, BLK, D_OUT), lambda i: (0, i, 0)),
        pl.BlockSpec((BLK, D_OUT), lambda i: (i, 0)),
        pl.BlockSpec((BLK, 1), lambda i: (i, 0)),
        pl.BlockSpec((BLK, 1), lambda i: (i, 0)),
        pl.BlockSpec((1, D_OUT), lambda i: (0, 0)),
        pl.BlockSpec((D_OUT, 1), lambda i: (0, 0)),
        pl.BlockSpec((1, 1), lambda i: (0, 0)),
    ],
    out_specs=pl.BlockSpec((G, 1), lambda i: (0, 0)),
    out_shape=jax.ShapeDtypeStruct((G, 1), jnp.float32),
    scratch_shapes=[
        pltpu.VMEM((G, D_OUT), jnp.float32),
        pltpu.VMEM((8, D_OUT), jnp.float32),
    ],
)


def kernel(x, edge_index, edge_attr, batch, W_conv, b_conv, W_lin, b_lin):
    src = edge_index[0].reshape(NW, NCHUNK, CHUNK)
    dst = edge_index[1].reshape(NW, NCHUNK, CHUNK)
    ew = edge_attr.reshape(NW, NCHUNK, CHUNK)

    xw, xw_bf = _mm_kernel(x, W_conv[:, _PERM])
    eh, dinv = _msg_kernel(xw_bf, src, dst, ew)

    out = _fin_kernel(
        eh, xw, dinv.reshape(NPAD, 1)[:N], batch.reshape(N, 1),
        b_conv[_PERM].reshape(1, D_OUT), W_lin[_PERM, :], b_lin.reshape(1, 1),
    )
    return out
